# baseline (speedup 1.0000x reference)
import jax
import jax.numpy as jnp
from jax import lax
from jax.experimental import pallas as pl
from jax.experimental.pallas import tpu as pltpu
from jax.experimental.pallas import tpu_sc as plsc

NUM_FIELDS = 26
VOCAB = 100000
EMBED_DIM = 32
BATCH = 16384

NC, NS, L = 2, 16, 16
NW = NC * NS
B_PER_W = BATCH // NW
CB = 128
NCH = B_PER_W // CB

VBLK = 512
NVB = (VOCAB + VBLK - 1) // VBLK
ROWS_FLAT = NUM_FIELDS * NVB * VBLK


def _tc_body(x_ref, o_ref):
    x = x_ref[0]
    o_ref[0] = jnp.concatenate(
        [jnp.transpose(x[:, q * 128:(q + 1) * 128]) for q in range(4)], axis=1
    )


def _detranspose(tables_t):
    return pl.pallas_call(
        _tc_body,
        grid=(NUM_FIELDS, NVB),
        in_specs=[pl.BlockSpec((1, EMBED_DIM, VBLK), lambda f, c: (f, 0, c))],
        out_specs=pl.BlockSpec((1, 128, 128), lambda f, c: (f, c, 0)),
        out_shape=jax.ShapeDtypeStruct((NUM_FIELDS, NVB * 128, 128), jnp.float32),
    )(tables_t)


def _sc_body(idx_hbm, tab_hbm, out_hbm, idxs_v, rlist_v, rows_v, sem_g, sem_o):
    wid = lax.axis_index("s") * NC + lax.axis_index("c")
    base_b = wid * B_PER_W

    def chunk_body(c, carry):
        b0 = base_b + c * CB
        pltpu.sync_copy(idx_hbm.at[:, pl.ds(b0, CB)], idxs_v)
        for f in range(NUM_FIELDS):
            for g in range(CB // L):
                v = idxs_v[f, pl.ds(g * L, L)]
                r = (((v >> 9) + jnp.int32(f * NVB)) * 512
                     + (v & 127) * 4 + ((v >> 7) & 3))
                rlist_v[f, pl.ds(g * L, L)] = r
        gathers = [
            pltpu.async_copy(tab_hbm.at[rlist_v.at[f]], rows_v.at[f], sem_g)
            for f in range(NUM_FIELDS)
        ]
        for cp in gathers:
            cp.wait()
        stores = [
            pltpu.async_copy(rows_v.at[f], out_hbm.at[pl.ds(b0, CB), f], sem_o)
            for f in range(NUM_FIELDS)
        ]
        for cp in stores:
            cp.wait()
        return carry

    lax.fori_loop(0, NCH, chunk_body, 0)


def kernel(indices, tables):
    idx = indices.astype(jnp.int32)
    tables_t = jnp.transpose(tables, (0, 2, 1))
    inter = _detranspose(tables_t)
    flat = inter.reshape(ROWS_FLAT, EMBED_DIM)
    mesh = plsc.VectorSubcoreMesh(
        core_axis_name="c", subcore_axis_name="s", num_cores=NC, num_subcores=NS
    )
    out = pl.kernel(
        _sc_body,
        out_type=jax.ShapeDtypeStruct((BATCH, NUM_FIELDS, EMBED_DIM), jnp.float32),
        mesh=mesh,
        compiler_params=pltpu.CompilerParams(
            needs_layout_passes=False, use_tc_tiling_on_sc=False
        ),
        scratch_types=[
            pltpu.VMEM((NUM_FIELDS, CB), jnp.int32),
            pltpu.VMEM((NUM_FIELDS, CB), jnp.int32),
            pltpu.VMEM((NUM_FIELDS, CB, EMBED_DIM), jnp.float32),
            pltpu.SemaphoreType.DMA,
            pltpu.SemaphoreType.DMA,
        ],
    )(idx, flat)
    return out


# speedup vs baseline: 3.0616x; 3.0616x over previous
import jax
import jax.numpy as jnp
from jax import lax
from jax.experimental import pallas as pl
from jax.experimental.pallas import tpu as pltpu
from jax.experimental.pallas import tpu_sc as plsc

NUM_FIELDS = 26
VOCAB = 100000
EMBED_DIM = 32
BATCH = 16384

NC, NS, L = 2, 16, 16
NW = NC * NS
B_PER_W = BATCH // NW
CB = 128
NCH = B_PER_W // CB

VBLK = 4096
NVB = (VOCAB + VBLK - 1) // VBLK
NVB512 = NVB * VBLK // 512
ROWS_FLAT = NUM_FIELDS * NVB * VBLK


def _tc_body(x_ref, o_ref):
    x = x_ref[0]
    eye = jnp.eye(EMBED_DIM, dtype=jnp.float32)
    dn = (((0,), (0,)), ((), ()))
    pieces = []
    for p in range(VBLK // 512):
        quads = [
            jax.lax.dot_general(
                x[:, p * 512 + q * 128:p * 512 + (q + 1) * 128], eye, dn,
                preferred_element_type=jnp.float32,
            )
            for q in range(4)
        ]
        pieces.append(jnp.concatenate(quads, axis=1))
    o_ref[0] = jnp.concatenate(pieces, axis=0)


def _detranspose(tables_t):
    return pl.pallas_call(
        _tc_body,
        grid=(NUM_FIELDS, NVB),
        in_specs=[pl.BlockSpec((1, EMBED_DIM, VBLK), lambda f, c: (f, 0, c))],
        out_specs=pl.BlockSpec((1, VBLK // 4, 128), lambda f, c: (f, c, 0)),
        out_shape=jax.ShapeDtypeStruct((NUM_FIELDS, NVB * VBLK // 4, 128), jnp.float32),
    )(tables_t)


def _sc_body(idx_hbm, tab_hbm, out_hbm, idxs_v, rlist_v, rows_v, sem_g, sem_o):
    wid = lax.axis_index("s") * NC + lax.axis_index("c")
    base_b = wid * B_PER_W

    def chunk_body(c, carry):
        b0 = base_b + c * CB
        pltpu.sync_copy(idx_hbm.at[:, pl.ds(b0, CB)], idxs_v)
        for f in range(NUM_FIELDS):
            for g in range(CB // L):
                v = idxs_v[f, pl.ds(g * L, L)]
                r = (((v >> 9) + jnp.int32(f * NVB512)) * 512
                     + (v & 127) * 4 + ((v >> 7) & 3))
                rlist_v[f, pl.ds(g * L, L)] = r
        gathers = [
            pltpu.async_copy(tab_hbm.at[rlist_v.at[f]], rows_v.at[f], sem_g)
            for f in range(NUM_FIELDS)
        ]
        for cp in gathers:
            cp.wait()
        stores = [
            pltpu.async_copy(rows_v.at[f], out_hbm.at[pl.ds(b0, CB), f], sem_o)
            for f in range(NUM_FIELDS)
        ]
        for cp in stores:
            cp.wait()
        return carry

    lax.fori_loop(0, NCH, chunk_body, 0)


def kernel(indices, tables):
    idx = indices.astype(jnp.int32)
    tables_t = jnp.transpose(tables, (0, 2, 1))
    inter = _detranspose(tables_t)
    flat = inter.reshape(ROWS_FLAT, EMBED_DIM)
    mesh = plsc.VectorSubcoreMesh(
        core_axis_name="c", subcore_axis_name="s", num_cores=NC, num_subcores=NS
    )
    out = pl.kernel(
        _sc_body,
        out_type=jax.ShapeDtypeStruct((BATCH, NUM_FIELDS, EMBED_DIM), jnp.float32),
        mesh=mesh,
        compiler_params=pltpu.CompilerParams(
            needs_layout_passes=False, use_tc_tiling_on_sc=False
        ),
        scratch_types=[
            pltpu.VMEM((NUM_FIELDS, CB), jnp.int32),
            pltpu.VMEM((NUM_FIELDS, CB), jnp.int32),
            pltpu.VMEM((NUM_FIELDS, CB, EMBED_DIM), jnp.float32),
            pltpu.SemaphoreType.DMA,
            pltpu.SemaphoreType.DMA,
        ],
    )(idx, flat)
    return out


# speedup vs baseline: 3.5221x; 1.1504x over previous
import jax
import jax.numpy as jnp
from jax import lax
from jax.experimental import pallas as pl
from jax.experimental.pallas import tpu as pltpu
from jax.experimental.pallas import tpu_sc as plsc

NUM_FIELDS = 26
VOCAB = 100000
EMBED_DIM = 32
BATCH = 16384

NC, NS, L = 2, 16, 16
NW = NC * NS
B_PER_W = BATCH // NW
CB = 128
NCH = B_PER_W // CB

VBLK = 4096
NVB = (VOCAB + VBLK - 1) // VBLK
NVB512 = NVB * VBLK // 512
ROWS_FLAT = NUM_FIELDS * NVB * VBLK


def _tc_body(x_ref, o_ref):
    x = x_ref[0]
    eye = jnp.eye(EMBED_DIM, dtype=jnp.float32)
    dn = (((0,), (0,)), ((), ()))
    for p in range(VBLK // 512):
        res = jax.lax.dot_general(
            x[:, p * 512:(p + 1) * 512], eye, dn,
            preferred_element_type=jnp.float32,
        )
        for q in range(4):
            o_ref[0, p * 128:(p + 1) * 128, q * 32:(q + 1) * 32] = (
                res[q * 128:(q + 1) * 128]
            )


def _detranspose(tables_t):
    return pl.pallas_call(
        _tc_body,
        grid=(NUM_FIELDS, NVB),
        in_specs=[pl.BlockSpec((1, EMBED_DIM, VBLK), lambda f, c: (f, 0, c))],
        out_specs=pl.BlockSpec((1, VBLK // 4, 128), lambda f, c: (f, c, 0)),
        out_shape=jax.ShapeDtypeStruct((NUM_FIELDS, NVB * VBLK // 4, 128), jnp.float32),
    )(tables_t)


def _sc_body(idx_hbm, tab_hbm, out_hbm, idxs_v, rlist_v, rows_v, sem_g, sem_o):
    wid = lax.axis_index("s") * NC + lax.axis_index("c")
    base_b = wid * B_PER_W

    def chunk_body(c, carry):
        b0 = base_b + c * CB
        pltpu.sync_copy(idx_hbm.at[:, pl.ds(b0, CB)], idxs_v)
        for f in range(NUM_FIELDS):
            for g in range(CB // L):
                v = idxs_v[f, pl.ds(g * L, L)]
                r = (((v >> 9) + jnp.int32(f * NVB512)) * 512
                     + (v & 127) * 4 + ((v >> 7) & 3))
                rlist_v[f, pl.ds(g * L, L)] = r
        gathers = [
            pltpu.async_copy(tab_hbm.at[rlist_v.at[f]], rows_v.at[f], sem_g)
            for f in range(NUM_FIELDS)
        ]
        for cp in gathers:
            cp.wait()
        stores = [
            pltpu.async_copy(
                rows_v.at[f],
                out_hbm.at[pl.ds(b0, CB), pl.ds(f * EMBED_DIM, EMBED_DIM)],
                sem_o,
            )
            for f in range(NUM_FIELDS)
        ]
        for cp in stores:
            cp.wait()
        return carry

    lax.fori_loop(0, NCH, chunk_body, 0)


def kernel(indices, tables):
    idx = indices.astype(jnp.int32)
    tables_t = jnp.transpose(tables, (0, 2, 1))
    inter = _detranspose(tables_t)
    flat = inter.reshape(ROWS_FLAT, EMBED_DIM)
    mesh = plsc.VectorSubcoreMesh(
        core_axis_name="c", subcore_axis_name="s", num_cores=NC, num_subcores=NS
    )
    out = pl.kernel(
        _sc_body,
        out_type=jax.ShapeDtypeStruct((BATCH, NUM_FIELDS * EMBED_DIM), jnp.float32),
        mesh=mesh,
        compiler_params=pltpu.CompilerParams(
            needs_layout_passes=False, use_tc_tiling_on_sc=False
        ),
        scratch_types=[
            pltpu.VMEM((NUM_FIELDS, CB), jnp.int32),
            pltpu.VMEM((NUM_FIELDS, CB), jnp.int32),
            pltpu.VMEM((NUM_FIELDS, CB, EMBED_DIM), jnp.float32),
            pltpu.SemaphoreType.DMA,
            pltpu.SemaphoreType.DMA,
        ],
    )(idx, flat)
    return out.reshape(BATCH, NUM_FIELDS, EMBED_DIM)
